# baseline (speedup 1.0000x reference)
import functools

import jax
import jax.numpy as jnp
from jax import lax
from jax.experimental import pallas as pl
from jax.experimental.pallas import tpu as pltpu
from jax.experimental.pallas import tpu_sc as plsc

N = 10000
R = 1805
NPAD = 10240
E = 320000
NC = 2
NS = 16
K = 128
NB = 8
CH = 160
CHD = CH // 2
EPAD = NS * CH * K
RPS = NPAD // NS
F32 = jnp.float32

_mesh = plsc.VectorSubcoreMesh(core_axis_name="c", subcore_axis_name="s")
_sc_params = pltpu.CompilerParams(use_tc_tiling_on_sc=False)


def _fill_vmem(ref, nrows, ncols, value):
    v = jnp.full((16,), value, F32)

    def row(i, carry):
        for j in range(ncols // 16):
            ref[i, pl.ds(j * 16, 16)] = v
        return carry

    lax.fori_loop(0, nrows, row, 0)


@functools.partial(
    pl.kernel,
    out_type=jax.ShapeDtypeStruct((NC, NPAD, 16), F32),
    mesh=_mesh,
    compiler_params=_sc_params,
    scratch_types=[
        pltpu.VMEM((CHD, K), jnp.int32),
        pltpu.VMEM((K, 16), F32),
        pltpu.VMEM((K, 16), F32),
        pltpu.VMEM_SHARED((NPAD, 16), F32),
    ],
)
def _sc_degree(dst_hbm, out_hbm, dstv, ones, zbuf, acc):
    c = lax.axis_index("c")
    s = lax.axis_index("s")
    pltpu.sync_copy(dst_hbm.at[s].at[pl.ds(c * CHD, CHD)], dstv)
    _fill_vmem(ones, K, 16, 1.0)
    _fill_vmem(zbuf, K, 16, 0.0)
    base = s * RPS

    def zc(i, carry):
        pltpu.sync_copy(zbuf, acc.at[pl.ds(base + i * K, K)])
        return carry

    lax.fori_loop(0, RPS // K, zc, 0)
    plsc.subcore_barrier()

    def body(i, carry):
        pltpu.sync_copy(ones, acc.at[dstv.at[i]], add=True)
        return carry

    lax.fori_loop(0, CHD, body, 0)
    plsc.subcore_barrier()
    pltpu.sync_copy(acc.at[pl.ds(base, RPS)], out_hbm.at[c].at[pl.ds(base, RPS)])


def _make_sc_aggregate(Dh):
    @functools.partial(
        pl.kernel,
        out_type=jax.ShapeDtypeStruct((NC, NPAD, Dh), F32),
        mesh=_mesh,
        compiler_params=_sc_params,
        scratch_types=(
            [pltpu.VMEM((CH, K), jnp.int32),
             pltpu.VMEM((CH, K), jnp.int32)]
            + [pltpu.VMEM((K, Dh), F32) for _ in range(4)]
            + [pltpu.VMEM_SHARED((NPAD, Dh), F32)]
            + [pltpu.SemaphoreType.DMA for _ in range(8)]
        ),
    )
    def agg(src_hbm, dst_hbm, g_hbm, out_hbm, srcv, dstv, *rest):
        bufs = rest[:4]
        acc = rest[4]
        sem_g = rest[5:9]
        sem_s = rest[9:13]
        c = lax.axis_index("c")
        s = lax.axis_index("s")
        gh = g_hbm.at[c]
        pltpu.sync_copy(src_hbm.at[s], srcv)
        pltpu.sync_copy(dst_hbm.at[s], dstv)
        base = s * RPS
        pltpu.sync_copy(gh.at[pl.ds(base, RPS)], acc.at[pl.ds(base, RPS)])
        plsc.subcore_barrier()

        def gather(i, b):
            pltpu.async_copy(gh.at[srcv.at[i]], bufs[b], sem_g[b])

        def gather_wait(i, b):
            pltpu.make_async_copy(gh.at[srcv.at[i]], bufs[b], sem_g[b]).wait()

        def scatter(i, b):
            pltpu.async_copy(bufs[b], acc.at[dstv.at[i]], sem_s[b], add=True)

        def scatter_wait(i, b):
            pltpu.make_async_copy(bufs[b], acc.at[dstv.at[i]], sem_s[b]).wait()

        gather(0, 0)
        gather(1, 1)
        gather_wait(0, 0)
        scatter(0, 0)
        gather(2, 2)
        gather_wait(1, 1)
        scatter(1, 1)
        gather(3, 3)

        def body(j, carry):
            i0 = 4 * j
            for b in range(4):
                i = i0 + b
                ii = i + 2
                gather_wait(ii, (b + 2) % 4)
                scatter(ii, (b + 2) % 4)
                scatter_wait(i, b)

                @pl.when(ii + 2 < CH)
                def _():
                    gather(ii + 2, b)
            return carry

        lax.fori_loop(0, (CH - 2) // 4, body, 0)
        gather_wait(CH - 2, (CH - 2) % 4)
        scatter(CH - 2, (CH - 2) % 4)
        gather_wait(CH - 1, (CH - 1) % 4)
        scatter(CH - 1, (CH - 1) % 4)
        scatter_wait(CH - 4, (CH - 4) % 4)
        scatter_wait(CH - 3, (CH - 3) % 4)
        scatter_wait(CH - 2, (CH - 2) % 4)
        scatter_wait(CH - 1, (CH - 1) % 4)
        plsc.subcore_barrier()
        pltpu.sync_copy(acc.at[pl.ds(base, RPS)],
                        out_hbm.at[c].at[pl.ds(base, RPS)])

    return agg


_sc_agg1 = _make_sc_aggregate(64)
_sc_agg2 = _make_sc_aggregate(32)


_BR = 512


def _dis_block(degp, i, br):
    deg = degp[0, :, 0:1] + degp[1, :, 0:1]
    rid = i * br + lax.broadcasted_iota(jnp.int32, (br, 1), 0)
    deg = deg + jnp.where(rid < N, 1.0, 0.0).astype(F32)
    return jnp.where(deg > 0, lax.rsqrt(deg), 0.0)


def _mm_scale_body(x_ref, w_ref, degp_ref, o_ref):
    i = pl.program_id(0)
    dis = _dis_block(degp_ref[...], i, _BR)
    g = jnp.dot(x_ref[...], w_ref[...], preferred_element_type=F32) * dis
    o_ref[0] = g[:, :64]
    o_ref[1] = g[:, 64:]


def _tc_mm_scale(x_pad, W1, degp):
    return pl.pallas_call(
        _mm_scale_body,
        grid=(NPAD // _BR,),
        in_specs=[
            pl.BlockSpec((_BR, 128), lambda i: (i, 0)),
            pl.BlockSpec((128, 128), lambda i: (0, 0)),
            pl.BlockSpec((2, _BR, 16), lambda i: (0, i, 0)),
        ],
        out_specs=pl.BlockSpec((2, _BR, 64), lambda i: (0, i, 0)),
        out_shape=jax.ShapeDtypeStruct((2, NPAD, 64), F32),
    )(x_pad, W1, degp)


def _layer_body(p_ref, degp_ref, b_ref, w_ref, o_ref):
    i = pl.program_id(0)
    dis = _dis_block(degp_ref[...], i, _BR)
    t = jnp.concatenate([p_ref[0], p_ref[1]], axis=1)
    pre = t * dis + b_ref[...]
    h = jnp.maximum(pre, 0.0)
    g2 = jnp.dot(h, w_ref[...], preferred_element_type=F32) * dis
    o_ref[0] = g2[:, :32]
    o_ref[1] = g2[:, 32:]


def _tc_layer(p1, degp, b1, W2):
    return pl.pallas_call(
        _layer_body,
        grid=(NPAD // _BR,),
        in_specs=[
            pl.BlockSpec((2, _BR, 64), lambda i: (0, i, 0)),
            pl.BlockSpec((2, _BR, 16), lambda i: (0, i, 0)),
            pl.BlockSpec((1, 128), lambda i: (0, 0)),
            pl.BlockSpec((128, 64), lambda i: (0, 0)),
        ],
        out_specs=pl.BlockSpec((2, _BR, 32), lambda i: (0, i, 0)),
        out_shape=jax.ShapeDtypeStruct((2, NPAD, 32), F32),
    )(p1, degp, b1, W2)


def _final_h_body(p_ref, degp_ref, b_ref, o_ref):
    i = pl.program_id(0)
    dis = _dis_block(degp_ref[...], i, _BR)
    t = jnp.concatenate([p_ref[0], p_ref[1]], axis=1)
    o_ref[...] = t * dis + b_ref[...]


def _tc_final_h(p2, degp, b2):
    return pl.pallas_call(
        _final_h_body,
        grid=(NPAD // _BR,),
        in_specs=[
            pl.BlockSpec((2, _BR, 32), lambda i: (0, i, 0)),
            pl.BlockSpec((2, _BR, 16), lambda i: (0, i, 0)),
            pl.BlockSpec((1, 64), lambda i: (0, 0)),
        ],
        out_specs=pl.BlockSpec((_BR, 64), lambda i: (i, 0)),
        out_shape=jax.ShapeDtypeStruct((NPAD, 64), F32),
    )(p2, degp, b2)


_BM = 256
_BN = 1024


def _nt_body(a_ref, b_ref, o_ref):
    o_ref[...] = lax.dot_general(
        a_ref[...], b_ref[...], (((1,), (1,)), ((), ())),
        preferred_element_type=F32)


def _tc_decoder(rna, drug):
    gm = pl.cdiv(R, _BM)
    gn = pl.cdiv(N - R, _BN)
    return pl.pallas_call(
        _nt_body,
        grid=(gm, gn),
        in_specs=[
            pl.BlockSpec((_BM, 64), lambda i, j: (i, 0)),
            pl.BlockSpec((_BN, 64), lambda i, j: (j, 0)),
        ],
        out_specs=pl.BlockSpec((_BM, _BN), lambda i, j: (i, j)),
        out_shape=jax.ShapeDtypeStruct((R, N - R), F32),
    )(rna, drug)


def kernel(x, edge_index, W1, b1, W2, b2):
    ei = edge_index.astype(jnp.int32)
    padfill = jnp.full((EPAD - E,), N, jnp.int32)
    src3 = jnp.concatenate([ei[0], padfill]).reshape(NS, CH, K)
    dst3 = jnp.concatenate([ei[1], padfill]).reshape(NS, CH, K)
    x_pad = jnp.pad(x, ((0, NPAD - N), (0, 0)))

    degp = _sc_degree(dst3)
    g1 = _tc_mm_scale(x_pad, W1, degp)
    p1 = _sc_agg1(src3, dst3, g1)
    g2 = _tc_layer(p1, degp, b1.reshape(1, 128), W2)
    p2 = _sc_agg2(src3, dst3, g2)
    h2 = _tc_final_h(p2, degp, b2.reshape(1, 64))

    rna = h2[:R]
    drug = h2[R:N]
    return _tc_decoder(rna, drug)


# speedup vs baseline: 1.0339x; 1.0339x over previous
import functools

import jax
import jax.numpy as jnp
from jax import lax
from jax.experimental import pallas as pl
from jax.experimental.pallas import tpu as pltpu
from jax.experimental.pallas import tpu_sc as plsc

N = 10000
R = 1805
NPAD = 10240
E = 320000
NC = 2
NS = 16
K = 128
NB = 8
CH = 160
CHD = CH // 2
EPAD = NS * CH * K
RPS = NPAD // NS
F32 = jnp.float32

_mesh = plsc.VectorSubcoreMesh(core_axis_name="c", subcore_axis_name="s")
_sc_params = pltpu.CompilerParams(use_tc_tiling_on_sc=False)


def _fill_vmem(ref, nrows, ncols, value):
    v = jnp.full((16,), value, F32)

    def row(i, carry):
        for j in range(ncols // 16):
            ref[i, pl.ds(j * 16, 16)] = v
        return carry

    lax.fori_loop(0, nrows, row, 0)


@functools.partial(
    pl.kernel,
    out_type=jax.ShapeDtypeStruct((NC, NPAD, 16), F32),
    mesh=_mesh,
    compiler_params=_sc_params,
    scratch_types=[
        pltpu.VMEM((CHD, K), jnp.int32),
        pltpu.VMEM((K, 16), F32),
        pltpu.VMEM((K, 16), F32),
        pltpu.VMEM_SHARED((NPAD, 16), F32),
    ],
)
def _sc_degree(dst_hbm, out_hbm, dstv, ones, zbuf, acc):
    c = lax.axis_index("c")
    s = lax.axis_index("s")
    pltpu.sync_copy(dst_hbm.at[s].at[pl.ds(c * CHD, CHD)], dstv)
    _fill_vmem(ones, K, 16, 1.0)
    _fill_vmem(zbuf, K, 16, 0.0)
    base = s * RPS

    def zc(i, carry):
        pltpu.sync_copy(zbuf, acc.at[pl.ds(base + i * K, K)])
        return carry

    lax.fori_loop(0, RPS // K, zc, 0)
    plsc.subcore_barrier()

    def body(i, carry):
        pltpu.sync_copy(ones, acc.at[dstv.at[i]], add=True)
        return carry

    lax.fori_loop(0, CHD, body, 0)
    plsc.subcore_barrier()
    pltpu.sync_copy(acc.at[pl.ds(base, RPS)], out_hbm.at[c].at[pl.ds(base, RPS)])


def _make_sc_aggregate(Dh):
    @functools.partial(
        pl.kernel,
        out_type=jax.ShapeDtypeStruct((NC, NPAD, Dh), F32),
        mesh=_mesh,
        compiler_params=_sc_params,
        scratch_types=[
            pltpu.VMEM((CH, K), jnp.int32),
            pltpu.VMEM((CH, K), jnp.int32),
            pltpu.VMEM((K, Dh), F32),
            pltpu.VMEM((K, Dh), F32),
            pltpu.VMEM((K, Dh), F32),
            pltpu.VMEM((K, Dh), F32),
            pltpu.VMEM_SHARED((NPAD, Dh), F32),
            pltpu.SemaphoreType.DMA,
            pltpu.SemaphoreType.DMA,
            pltpu.SemaphoreType.DMA,
            pltpu.SemaphoreType.DMA,
        ],
    )
    def agg(src_hbm, dst_hbm, g_hbm, out_hbm, srcv, dstv, b0, b1, b2, b3, acc,
            s0, s1, s2, s3):
        bufs = (b0, b1, b2, b3)
        sems = (s0, s1, s2, s3)
        c = lax.axis_index("c")
        s = lax.axis_index("s")
        gh = g_hbm.at[c]
        pltpu.sync_copy(src_hbm.at[s], srcv)
        pltpu.sync_copy(dst_hbm.at[s], dstv)
        base = s * RPS
        pltpu.sync_copy(gh.at[pl.ds(base, RPS)], acc.at[pl.ds(base, RPS)])
        plsc.subcore_barrier()

        def gather(i, b):
            pltpu.async_copy(gh.at[srcv.at[i]], bufs[b], sems[b])

        def gather_wait(i, b):
            pltpu.make_async_copy(gh.at[srcv.at[i]], bufs[b], sems[b]).wait()

        gather(0, 0)
        gather(1, 1)
        gather(2, 2)

        def body(j, carry):
            i0 = 4 * j
            for b in range(4):
                i = i0 + b

                @pl.when(i + 3 < CH)
                def _():
                    gather(i + 3, (b + 3) % 4)

                gather_wait(i, b)
                pltpu.sync_copy(bufs[b], acc.at[dstv.at[i]], add=True)
            return carry

        lax.fori_loop(0, CH // 4, body, 0)
        plsc.subcore_barrier()
        pltpu.sync_copy(acc.at[pl.ds(base, RPS)],
                        out_hbm.at[c].at[pl.ds(base, RPS)])

    return agg


_sc_agg1 = _make_sc_aggregate(64)
_sc_agg2 = _make_sc_aggregate(32)


_BR = 512


def _dis_block(degp, i, br):
    deg = degp[0, :, 0:1] + degp[1, :, 0:1]
    rid = i * br + lax.broadcasted_iota(jnp.int32, (br, 1), 0)
    deg = deg + jnp.where(rid < N, 1.0, 0.0).astype(F32)
    return jnp.where(deg > 0, lax.rsqrt(deg), 0.0)


def _mm_body(x_ref, w_ref, o_ref):
    o_ref[...] = jnp.dot(x_ref[...], w_ref[...], preferred_element_type=F32)


def _tc_matmul(x_pad, W1):
    return pl.pallas_call(
        _mm_body,
        grid=(NPAD // _BR,),
        in_specs=[
            pl.BlockSpec((_BR, 128), lambda i: (i, 0)),
            pl.BlockSpec((128, 128), lambda i: (0, 0)),
        ],
        out_specs=pl.BlockSpec((_BR, 128), lambda i: (i, 0)),
        out_shape=jax.ShapeDtypeStruct((NPAD, 128), F32),
    )(x_pad, W1)


def _scale_body(h_ref, degp_ref, o_ref):
    i = pl.program_id(0)
    dis = _dis_block(degp_ref[...], i, _BR)
    g = h_ref[...] * dis
    o_ref[0] = g[:, :64]
    o_ref[1] = g[:, 64:]


def _tc_scale(h1raw, degp):
    return pl.pallas_call(
        _scale_body,
        grid=(NPAD // _BR,),
        in_specs=[
            pl.BlockSpec((_BR, 128), lambda i: (i, 0)),
            pl.BlockSpec((2, _BR, 16), lambda i: (0, i, 0)),
        ],
        out_specs=pl.BlockSpec((2, _BR, 64), lambda i: (0, i, 0)),
        out_shape=jax.ShapeDtypeStruct((2, NPAD, 64), F32),
    )(h1raw, degp)


def _layer_body(p_ref, degp_ref, b_ref, w_ref, o_ref):
    i = pl.program_id(0)
    dis = _dis_block(degp_ref[...], i, _BR)
    t = jnp.concatenate([p_ref[0], p_ref[1]], axis=1)
    pre = t * dis + b_ref[...]
    h = jnp.maximum(pre, 0.0)
    g2 = jnp.dot(h, w_ref[...], preferred_element_type=F32) * dis
    o_ref[0] = g2[:, :32]
    o_ref[1] = g2[:, 32:]


def _tc_layer(p1, degp, b1, W2):
    return pl.pallas_call(
        _layer_body,
        grid=(NPAD // _BR,),
        in_specs=[
            pl.BlockSpec((2, _BR, 64), lambda i: (0, i, 0)),
            pl.BlockSpec((2, _BR, 16), lambda i: (0, i, 0)),
            pl.BlockSpec((1, 128), lambda i: (0, 0)),
            pl.BlockSpec((128, 64), lambda i: (0, 0)),
        ],
        out_specs=pl.BlockSpec((2, _BR, 32), lambda i: (0, i, 0)),
        out_shape=jax.ShapeDtypeStruct((2, NPAD, 32), F32),
    )(p1, degp, b1, W2)


def _final_h_body(p_ref, degp_ref, b_ref, o_ref):
    i = pl.program_id(0)
    dis = _dis_block(degp_ref[...], i, _BR)
    t = jnp.concatenate([p_ref[0], p_ref[1]], axis=1)
    o_ref[...] = t * dis + b_ref[...]


def _tc_final_h(p2, degp, b2):
    return pl.pallas_call(
        _final_h_body,
        grid=(NPAD // _BR,),
        in_specs=[
            pl.BlockSpec((2, _BR, 32), lambda i: (0, i, 0)),
            pl.BlockSpec((2, _BR, 16), lambda i: (0, i, 0)),
            pl.BlockSpec((1, 64), lambda i: (0, 0)),
        ],
        out_specs=pl.BlockSpec((_BR, 64), lambda i: (i, 0)),
        out_shape=jax.ShapeDtypeStruct((NPAD, 64), F32),
    )(p2, degp, b2)


_BM = 256
_BN = 1024


def _nt_body(a_ref, b_ref, o_ref):
    o_ref[...] = lax.dot_general(
        a_ref[...], b_ref[...], (((1,), (1,)), ((), ())),
        preferred_element_type=F32)


def _tc_decoder(rna, drug):
    gm = pl.cdiv(R, _BM)
    gn = pl.cdiv(N - R, _BN)
    return pl.pallas_call(
        _nt_body,
        grid=(gm, gn),
        in_specs=[
            pl.BlockSpec((_BM, 64), lambda i, j: (i, 0)),
            pl.BlockSpec((_BN, 64), lambda i, j: (j, 0)),
        ],
        out_specs=pl.BlockSpec((_BM, _BN), lambda i, j: (i, j)),
        out_shape=jax.ShapeDtypeStruct((R, N - R), F32),
    )(rna, drug)


def kernel(x, edge_index, W1, b1, W2, b2):
    ei = edge_index.astype(jnp.int32)
    padfill = jnp.full((EPAD - E,), N, jnp.int32)
    src3 = jnp.concatenate([ei[0], padfill]).reshape(NS, CH, K)
    dst3 = jnp.concatenate([ei[1], padfill]).reshape(NS, CH, K)
    x_pad = jnp.pad(x, ((0, NPAD - N), (0, 0)))

    degp = _sc_degree(dst3)
    h1raw = _tc_matmul(x_pad, W1)
    g1 = _tc_scale(h1raw, degp)
    p1 = _sc_agg1(src3, dst3, g1)
    g2 = _tc_layer(p1, degp, b1.reshape(1, 128), W2)
    p2 = _sc_agg2(src3, dst3, g2)
    h2 = _tc_final_h(p2, degp, b2.reshape(1, 64))

    rna = h2[:R]
    drug = h2[R:N]
    return _tc_decoder(rna, drug)


# speedup vs baseline: 1.2779x; 1.2360x over previous
import functools

import jax
import jax.numpy as jnp
from jax import lax
from jax.experimental import pallas as pl
from jax.experimental.pallas import tpu as pltpu
from jax.experimental.pallas import tpu_sc as plsc

N = 10000
R = 1805
NPAD = 10240
E = 320000
NC = 2
NS = 16
K = 128
NB = 8
CH = 158
CHD = CH // 2
EPAD = NS * CH * K
RPS = NPAD // NS
F32 = jnp.float32

_mesh = plsc.VectorSubcoreMesh(core_axis_name="c", subcore_axis_name="s")
_sc_params = pltpu.CompilerParams(use_tc_tiling_on_sc=False)


def _fill_vmem(ref, nrows, ncols, value):
    v = jnp.full((16,), value, F32)

    def row(i, carry):
        for j in range(ncols // 16):
            ref[i, pl.ds(j * 16, 16)] = v
        return carry

    lax.fori_loop(0, nrows, row, 0)


@functools.partial(
    pl.kernel,
    out_type=jax.ShapeDtypeStruct((NC, NPAD, 16), F32),
    mesh=_mesh,
    compiler_params=_sc_params,
    scratch_types=[
        pltpu.VMEM((CHD, K), jnp.int32),
        pltpu.VMEM((K, 16), F32),
        pltpu.VMEM((K, 16), F32),
        pltpu.VMEM_SHARED((NPAD, 16), F32),
    ],
)
def _sc_degree(dst_hbm, out_hbm, dstv, ones, zbuf, acc):
    c = lax.axis_index("c")
    s = lax.axis_index("s")
    pltpu.sync_copy(dst_hbm.at[s].at[pl.ds(c * CHD, CHD)], dstv)
    _fill_vmem(ones, K, 16, 1.0)
    _fill_vmem(zbuf, K, 16, 0.0)
    base = s * RPS

    def zc(i, carry):
        pltpu.sync_copy(zbuf, acc.at[pl.ds(base + i * K, K)])
        return carry

    lax.fori_loop(0, RPS // K, zc, 0)
    plsc.subcore_barrier()

    def body(i, carry):
        pltpu.sync_copy(ones, acc.at[dstv.at[i]], add=True)
        return carry

    lax.fori_loop(0, CHD, body, 0)
    plsc.subcore_barrier()
    pltpu.sync_copy(acc.at[pl.ds(base, RPS)], out_hbm.at[c].at[pl.ds(base, RPS)])


def _make_sc_aggregate(Dh):
    @functools.partial(
        pl.kernel,
        out_type=jax.ShapeDtypeStruct((NC, NPAD, Dh), F32),
        mesh=_mesh,
        compiler_params=_sc_params,
        scratch_types=[
            pltpu.VMEM((CH, K), jnp.int32),
            pltpu.VMEM((CH, K), jnp.int32),
            pltpu.VMEM((K, Dh), F32),
            pltpu.VMEM((K, Dh), F32),
            pltpu.VMEM_SHARED((NPAD, Dh), F32),
            pltpu.SemaphoreType.DMA,
            pltpu.SemaphoreType.DMA,
        ],
    )
    def agg(src_hbm, dst_hbm, g_hbm, out_hbm, srcv, dstv, rows_a, rows_b, acc,
            sem_a, sem_b):
        c = lax.axis_index("c")
        s = lax.axis_index("s")
        gh = g_hbm.at[c]
        pltpu.sync_copy(src_hbm.at[s], srcv)
        pltpu.sync_copy(dst_hbm.at[s], dstv)
        base = s * RPS
        pltpu.sync_copy(gh.at[pl.ds(base, RPS)], acc.at[pl.ds(base, RPS)])
        plsc.subcore_barrier()

        pltpu.async_copy(gh.at[srcv.at[0]], rows_a, sem_a)

        def body(j, carry):
            i0 = 2 * j
            pltpu.async_copy(gh.at[srcv.at[i0 + 1]], rows_b, sem_b)
            pltpu.make_async_copy(gh.at[srcv.at[i0]], rows_a, sem_a).wait()
            pltpu.sync_copy(rows_a, acc.at[dstv.at[i0]], add=True)

            @pl.when(j < CH // 2 - 1)
            def _():
                pltpu.async_copy(gh.at[srcv.at[i0 + 2]], rows_a, sem_a)

            pltpu.make_async_copy(gh.at[srcv.at[i0 + 1]], rows_b, sem_b).wait()
            pltpu.sync_copy(rows_b, acc.at[dstv.at[i0 + 1]], add=True)
            return carry

        lax.fori_loop(0, CH // 2, body, 0)
        plsc.subcore_barrier()
        pltpu.sync_copy(acc.at[pl.ds(base, RPS)],
                        out_hbm.at[c].at[pl.ds(base, RPS)])

    return agg


_sc_agg1 = _make_sc_aggregate(64)
_sc_agg2 = _make_sc_aggregate(32)


_BR = 512


def _dis_block(degp, i, br):
    deg = degp[0, :, 0:1] + degp[1, :, 0:1]
    rid = i * br + lax.broadcasted_iota(jnp.int32, (br, 1), 0)
    deg = deg + jnp.where(rid < N, 1.0, 0.0).astype(F32)
    return jnp.where(deg > 0, lax.rsqrt(deg), 0.0)


def _mm_scale_body(x_ref, w_ref, degp_ref, o_ref):
    i = pl.program_id(0)
    dis = _dis_block(degp_ref[...], i, _BR)
    g = jnp.dot(x_ref[...], w_ref[...], preferred_element_type=F32) * dis
    o_ref[0] = g[:, :64]
    o_ref[1] = g[:, 64:]


def _tc_mm_scale(x_pad, W1, degp):
    return pl.pallas_call(
        _mm_scale_body,
        grid=(NPAD // _BR,),
        in_specs=[
            pl.BlockSpec((_BR, 128), lambda i: (i, 0)),
            pl.BlockSpec((128, 128), lambda i: (0, 0)),
            pl.BlockSpec((2, _BR, 16), lambda i: (0, i, 0)),
        ],
        out_specs=pl.BlockSpec((2, _BR, 64), lambda i: (0, i, 0)),
        out_shape=jax.ShapeDtypeStruct((2, NPAD, 64), F32),
    )(x_pad, W1, degp)


def _layer_body(p_ref, degp_ref, b_ref, w_ref, o_ref):
    i = pl.program_id(0)
    dis = _dis_block(degp_ref[...], i, _BR)
    t = jnp.concatenate([p_ref[0], p_ref[1]], axis=1)
    pre = t * dis + b_ref[...]
    h = jnp.maximum(pre, 0.0)
    g2 = jnp.dot(h, w_ref[...], preferred_element_type=F32) * dis
    o_ref[0] = g2[:, :32]
    o_ref[1] = g2[:, 32:]


def _tc_layer(p1, degp, b1, W2):
    return pl.pallas_call(
        _layer_body,
        grid=(NPAD // _BR,),
        in_specs=[
            pl.BlockSpec((2, _BR, 64), lambda i: (0, i, 0)),
            pl.BlockSpec((2, _BR, 16), lambda i: (0, i, 0)),
            pl.BlockSpec((1, 128), lambda i: (0, 0)),
            pl.BlockSpec((128, 64), lambda i: (0, 0)),
        ],
        out_specs=pl.BlockSpec((2, _BR, 32), lambda i: (0, i, 0)),
        out_shape=jax.ShapeDtypeStruct((2, NPAD, 32), F32),
    )(p1, degp, b1, W2)


def _final_h_body(p_ref, degp_ref, b_ref, o_ref):
    i = pl.program_id(0)
    dis = _dis_block(degp_ref[...], i, _BR)
    t = jnp.concatenate([p_ref[0], p_ref[1]], axis=1)
    o_ref[...] = t * dis + b_ref[...]


def _tc_final_h(p2, degp, b2):
    return pl.pallas_call(
        _final_h_body,
        grid=(NPAD // _BR,),
        in_specs=[
            pl.BlockSpec((2, _BR, 32), lambda i: (0, i, 0)),
            pl.BlockSpec((2, _BR, 16), lambda i: (0, i, 0)),
            pl.BlockSpec((1, 64), lambda i: (0, 0)),
        ],
        out_specs=pl.BlockSpec((_BR, 64), lambda i: (i, 0)),
        out_shape=jax.ShapeDtypeStruct((NPAD, 64), F32),
    )(p2, degp, b2)


_BM = 512
_BN = 2048


def _nt_body(a_ref, b_ref, o_ref):
    o_ref[...] = lax.dot_general(
        a_ref[...], b_ref[...], (((1,), (1,)), ((), ())),
        preferred_element_type=F32)


def _tc_decoder(rna, drug):
    gm = pl.cdiv(R, _BM)
    gn = pl.cdiv(N - R, _BN)
    return pl.pallas_call(
        _nt_body,
        grid=(gm, gn),
        in_specs=[
            pl.BlockSpec((_BM, 64), lambda i, j: (i, 0)),
            pl.BlockSpec((_BN, 64), lambda i, j: (j, 0)),
        ],
        out_specs=pl.BlockSpec((_BM, _BN), lambda i, j: (i, j)),
        out_shape=jax.ShapeDtypeStruct((R, N - R), F32),
    )(rna, drug)


def kernel(x, edge_index, W1, b1, W2, b2):
    ei = edge_index.astype(jnp.int32)
    padfill = jnp.full((EPAD - E,), N, jnp.int32)
    src3 = jnp.concatenate([ei[0], padfill]).reshape(NS, CH, K)
    dst3 = jnp.concatenate([ei[1], padfill]).reshape(NS, CH, K)
    x_pad = jnp.pad(x, ((0, NPAD - N), (0, 0)))

    degp = _sc_degree(dst3)
    g1 = _tc_mm_scale(x_pad, W1, degp)
    p1 = _sc_agg1(src3, dst3, g1)
    g2 = _tc_layer(p1, degp, b1.reshape(1, 128), W2)
    p2 = _sc_agg2(src3, dst3, g2)
    h2 = _tc_final_h(p2, degp, b2.reshape(1, 64))

    rna = h2[:R]
    drug = h2[R:N]
    return _tc_decoder(rna, drug)


# speedup vs baseline: 1.2822x; 1.0033x over previous
import functools

import jax
import jax.numpy as jnp
from jax import lax
from jax.experimental import pallas as pl
from jax.experimental.pallas import tpu as pltpu
from jax.experimental.pallas import tpu_sc as plsc

N = 10000
R = 1805
NPAD = 10240
E = 320000
NC = 2
NS = 16
K = 128
NB = 8
CH = 158
CHD = CH // 2
EPAD = NS * CH * K
RPS = NPAD // NS
F32 = jnp.float32

_mesh = plsc.VectorSubcoreMesh(core_axis_name="c", subcore_axis_name="s")
_sc_params = pltpu.CompilerParams(use_tc_tiling_on_sc=False)


def _fill_vmem(ref, nrows, ncols, value):
    v = jnp.full((16,), value, F32)

    def row(i, carry):
        for j in range(ncols // 16):
            ref[i, pl.ds(j * 16, 16)] = v
        return carry

    lax.fori_loop(0, nrows, row, 0)


@functools.partial(
    pl.kernel,
    out_type=jax.ShapeDtypeStruct((NC, NPAD, 16), F32),
    mesh=_mesh,
    compiler_params=_sc_params,
    scratch_types=[
        pltpu.VMEM((CHD, K), jnp.int32),
        pltpu.VMEM((K, 16), F32),
        pltpu.VMEM((K, 16), F32),
        pltpu.VMEM_SHARED((NPAD, 16), F32),
    ],
)
def _sc_degree(dst_hbm, out_hbm, dstv, ones, zbuf, acc):
    c = lax.axis_index("c")
    s = lax.axis_index("s")
    pltpu.sync_copy(dst_hbm.at[s].at[pl.ds(c * CHD, CHD)], dstv)
    _fill_vmem(ones, K, 16, 1.0)
    _fill_vmem(zbuf, K, 16, 0.0)
    base = s * RPS

    def zc(i, carry):
        pltpu.sync_copy(zbuf, acc.at[pl.ds(base + i * K, K)])
        return carry

    lax.fori_loop(0, RPS // K, zc, 0)
    plsc.subcore_barrier()

    def body(i, carry):
        pltpu.sync_copy(ones, acc.at[dstv.at[i]], add=True)
        return carry

    lax.fori_loop(0, CHD, body, 0)
    plsc.subcore_barrier()
    pltpu.sync_copy(acc.at[pl.ds(base, RPS)], out_hbm.at[c].at[pl.ds(base, RPS)])


def _make_sc_aggregate(Dh):
    @functools.partial(
        pl.kernel,
        out_type=jax.ShapeDtypeStruct((NC, NPAD, Dh), F32),
        mesh=_mesh,
        compiler_params=_sc_params,
        scratch_types=[
            pltpu.VMEM((CH, K), jnp.int32),
            pltpu.VMEM((CH, K), jnp.int32),
            pltpu.VMEM((K, Dh), F32),
            pltpu.VMEM((K, Dh), F32),
            pltpu.VMEM_SHARED((NPAD, Dh), F32),
            pltpu.SemaphoreType.DMA,
            pltpu.SemaphoreType.DMA,
        ],
    )
    def agg(src_hbm, dst_hbm, g_hbm, out_hbm, srcv, dstv, rows_a, rows_b, acc,
            sem_a, sem_b):
        c = lax.axis_index("c")
        s = lax.axis_index("s")
        gh = g_hbm.at[c]
        pltpu.sync_copy(src_hbm.at[s], srcv)
        pltpu.sync_copy(dst_hbm.at[s], dstv)
        base = s * RPS
        pltpu.sync_copy(gh.at[pl.ds(base, RPS)], acc.at[pl.ds(base, RPS)])
        plsc.subcore_barrier()

        pltpu.async_copy(gh.at[srcv.at[0]], rows_a, sem_a)

        def body(j, carry):
            i0 = 2 * j
            pltpu.async_copy(gh.at[srcv.at[i0 + 1]], rows_b, sem_b)
            pltpu.make_async_copy(gh.at[srcv.at[i0]], rows_a, sem_a).wait()
            pltpu.sync_copy(rows_a, acc.at[dstv.at[i0]], add=True)

            @pl.when(j < CH // 2 - 1)
            def _():
                pltpu.async_copy(gh.at[srcv.at[i0 + 2]], rows_a, sem_a)

            pltpu.make_async_copy(gh.at[srcv.at[i0 + 1]], rows_b, sem_b).wait()
            pltpu.sync_copy(rows_b, acc.at[dstv.at[i0 + 1]], add=True)
            return carry

        lax.fori_loop(0, CH // 2, body, 0)
        plsc.subcore_barrier()
        pltpu.sync_copy(acc.at[pl.ds(base, RPS)],
                        out_hbm.at[c].at[pl.ds(base, RPS)])

    return agg


_sc_agg1 = _make_sc_aggregate(64)
_sc_agg2 = _make_sc_aggregate(32)


_BR = 512


def _dis_block(degp, i, br):
    deg = degp[0, :, 0:1] + degp[1, :, 0:1]
    rid = i * br + lax.broadcasted_iota(jnp.int32, (br, 1), 0)
    deg = deg + jnp.where(rid < N, 1.0, 0.0).astype(F32)
    return jnp.where(deg > 0, lax.rsqrt(deg), 0.0)


def _mm_scale_body(x_ref, w_ref, degp_ref, o_ref):
    i = pl.program_id(0)
    dis = _dis_block(degp_ref[...], i, _BR)
    g = jnp.dot(x_ref[...], w_ref[...], preferred_element_type=F32) * dis
    o_ref[0] = g[:, :64]
    o_ref[1] = g[:, 64:]


def _tc_mm_scale(x_pad, W1, degp):
    return pl.pallas_call(
        _mm_scale_body,
        grid=(NPAD // _BR,),
        in_specs=[
            pl.BlockSpec((_BR, 128), lambda i: (i, 0)),
            pl.BlockSpec((128, 128), lambda i: (0, 0)),
            pl.BlockSpec((2, _BR, 16), lambda i: (0, i, 0)),
        ],
        out_specs=pl.BlockSpec((2, _BR, 64), lambda i: (0, i, 0)),
        out_shape=jax.ShapeDtypeStruct((2, NPAD, 64), F32),
    )(x_pad, W1, degp)


def _layer_body(p_ref, degp_ref, b_ref, w_ref, o_ref):
    i = pl.program_id(0)
    dis = _dis_block(degp_ref[...], i, _BR)
    ha = jnp.maximum(p_ref[0] * dis + b_ref[0, :, :64], 0.0)
    hb = jnp.maximum(p_ref[1] * dis + b_ref[0, :, 64:], 0.0)
    g2 = (jnp.dot(ha, w_ref[0], preferred_element_type=F32)
          + jnp.dot(hb, w_ref[1], preferred_element_type=F32)) * dis
    o_ref[0] = g2[:, :32]
    o_ref[1] = g2[:, 32:]


def _tc_layer(p1, degp, b1, W2):
    return pl.pallas_call(
        _layer_body,
        grid=(NPAD // _BR,),
        in_specs=[
            pl.BlockSpec((2, _BR, 64), lambda i: (0, i, 0)),
            pl.BlockSpec((2, _BR, 16), lambda i: (0, i, 0)),
            pl.BlockSpec((1, 1, 128), lambda i: (0, 0, 0)),
            pl.BlockSpec((2, 64, 64), lambda i: (0, 0, 0)),
        ],
        out_specs=pl.BlockSpec((2, _BR, 32), lambda i: (0, i, 0)),
        out_shape=jax.ShapeDtypeStruct((2, NPAD, 32), F32),
    )(p1, degp, b1, W2)


def _final_h_body(p_ref, degp_ref, b_ref, o_ref):
    i = pl.program_id(0)
    dis = _dis_block(degp_ref[...], i, _BR)
    o_ref[:, :32] = p_ref[0] * dis + b_ref[0, :, :32]
    o_ref[:, 32:] = p_ref[1] * dis + b_ref[0, :, 32:]


def _tc_final_h(p2, degp, b2):
    return pl.pallas_call(
        _final_h_body,
        grid=(NPAD // _BR,),
        in_specs=[
            pl.BlockSpec((2, _BR, 32), lambda i: (0, i, 0)),
            pl.BlockSpec((2, _BR, 16), lambda i: (0, i, 0)),
            pl.BlockSpec((1, 1, 64), lambda i: (0, 0, 0)),
        ],
        out_specs=pl.BlockSpec((_BR, 64), lambda i: (i, 0)),
        out_shape=jax.ShapeDtypeStruct((NPAD, 64), F32),
    )(p2, degp, b2)


_BM = 512
_BN = 4096


def _nt_body(a_ref, b_ref, o_ref):
    o_ref[...] = lax.dot_general(
        a_ref[...], b_ref[...], (((1,), (1,)), ((), ())),
        preferred_element_type=F32)


def _tc_decoder(rna, drug):
    gm = pl.cdiv(R, _BM)
    gn = pl.cdiv(N - R, _BN)
    return pl.pallas_call(
        _nt_body,
        grid=(gm, gn),
        in_specs=[
            pl.BlockSpec((_BM, 64), lambda i, j: (i, 0)),
            pl.BlockSpec((_BN, 64), lambda i, j: (j, 0)),
        ],
        out_specs=pl.BlockSpec((_BM, _BN), lambda i, j: (i, j)),
        out_shape=jax.ShapeDtypeStruct((R, N - R), F32),
    )(rna, drug)


def kernel(x, edge_index, W1, b1, W2, b2):
    ei = edge_index.astype(jnp.int32)
    padfill = jnp.full((EPAD - E,), N, jnp.int32)
    src3 = jnp.concatenate([ei[0], padfill]).reshape(NS, CH, K)
    dst3 = jnp.concatenate([ei[1], padfill]).reshape(NS, CH, K)
    x_pad = jnp.pad(x, ((0, NPAD - N), (0, 0)))

    degp = _sc_degree(dst3)
    g1 = _tc_mm_scale(x_pad, W1, degp)
    p1 = _sc_agg1(src3, dst3, g1)
    g2 = _tc_layer(p1, degp, b1.reshape(1, 1, 128), W2.reshape(2, 64, 64))
    p2 = _sc_agg2(src3, dst3, g2)
    h2 = _tc_final_h(p2, degp, b2.reshape(1, 1, 64))

    rna = h2[:R]
    drug = h2[R:N]
    return _tc_decoder(rna, drug)


# speedup vs baseline: 1.3245x; 1.0331x over previous
import functools

import jax
import jax.numpy as jnp
from jax import lax
from jax.experimental import pallas as pl
from jax.experimental.pallas import tpu as pltpu
from jax.experimental.pallas import tpu_sc as plsc

N = 10000
R = 1805
NPAD = 10240
E = 320000
NC = 2
NS = 16
K = 128
NB = 8
CH = 158
CHD = CH // 2
EPAD = NS * CH * K
RPS = NPAD // NS
F32 = jnp.float32

_mesh = plsc.VectorSubcoreMesh(core_axis_name="c", subcore_axis_name="s")
_sc_params = pltpu.CompilerParams(use_tc_tiling_on_sc=False)


def _fill_vmem(ref, nrows, ncols, value):
    v = jnp.full((16,), value, F32)

    def row(i, carry):
        for j in range(ncols // 16):
            ref[i, pl.ds(j * 16, 16)] = v
        return carry

    lax.fori_loop(0, nrows, row, 0)


@functools.partial(
    pl.kernel,
    out_type=jax.ShapeDtypeStruct((NC, NPAD, 16), F32),
    mesh=_mesh,
    compiler_params=_sc_params,
    scratch_types=[
        pltpu.VMEM((CHD, K), jnp.int32),
        pltpu.VMEM((K, 16), F32),
        pltpu.VMEM((K, 16), F32),
        pltpu.VMEM_SHARED((NPAD, 16), F32),
    ],
)
def _sc_degree(dst_hbm, out_hbm, dstv, ones, zbuf, acc):
    c = lax.axis_index("c")
    s = lax.axis_index("s")
    pltpu.sync_copy(dst_hbm.at[s].at[pl.ds(c * CHD, CHD)], dstv)
    _fill_vmem(ones, K, 16, 1.0)
    _fill_vmem(zbuf, K, 16, 0.0)
    base = s * RPS

    def zc(i, carry):
        pltpu.sync_copy(zbuf, acc.at[pl.ds(base + i * K, K)])
        return carry

    lax.fori_loop(0, RPS // K, zc, 0)
    plsc.subcore_barrier()

    def body(i, carry):
        pltpu.sync_copy(ones, acc.at[dstv.at[i]], add=True)
        return carry

    lax.fori_loop(0, CHD, body, 0)
    plsc.subcore_barrier()
    pltpu.sync_copy(acc.at[pl.ds(base, RPS)], out_hbm.at[c].at[pl.ds(base, RPS)])


def _make_sc_aggregate(Dh):
    @functools.partial(
        pl.kernel,
        out_type=jax.ShapeDtypeStruct((NC, NPAD, Dh), F32),
        mesh=_mesh,
        compiler_params=_sc_params,
        scratch_types=[
            pltpu.VMEM((CH, K), jnp.int32),
            pltpu.VMEM((CH, K), jnp.int32),
            pltpu.VMEM((K, Dh), F32),
            pltpu.VMEM((K, Dh), F32),
            pltpu.VMEM_SHARED((NPAD, Dh), F32),
            pltpu.SemaphoreType.DMA,
            pltpu.SemaphoreType.DMA,
        ],
    )
    def agg(src_hbm, dst_hbm, g_hbm, out_hbm, srcv, dstv, rows_a, rows_b, acc,
            sem_a, sem_b):
        c = lax.axis_index("c")
        s = lax.axis_index("s")
        gh = g_hbm.at[c]
        pltpu.sync_copy(src_hbm.at[s], srcv)
        pltpu.sync_copy(dst_hbm.at[s], dstv)
        base = s * RPS
        pltpu.sync_copy(gh.at[pl.ds(base, RPS)], acc.at[pl.ds(base, RPS)])
        plsc.subcore_barrier()

        pltpu.async_copy(gh.at[srcv.at[0]], rows_a, sem_a)

        def body(j, carry):
            i0 = 2 * j
            pltpu.async_copy(gh.at[srcv.at[i0 + 1]], rows_b, sem_b)
            pltpu.make_async_copy(gh.at[srcv.at[i0]], rows_a, sem_a).wait()
            pltpu.sync_copy(rows_a, acc.at[dstv.at[i0]], add=True)

            @pl.when(j < CH // 2 - 1)
            def _():
                pltpu.async_copy(gh.at[srcv.at[i0 + 2]], rows_a, sem_a)

            pltpu.make_async_copy(gh.at[srcv.at[i0 + 1]], rows_b, sem_b).wait()
            pltpu.sync_copy(rows_b, acc.at[dstv.at[i0 + 1]], add=True)
            return carry

        lax.fori_loop(0, CH // 2, body, 0)
        plsc.subcore_barrier()
        pltpu.sync_copy(acc.at[pl.ds(base, RPS)],
                        out_hbm.at[c].at[pl.ds(base, RPS)])

    return agg


_sc_agg1 = _make_sc_aggregate(64)
_sc_agg2 = _make_sc_aggregate(32)


_BR = 1024


def _dis_block(degp, i, br):
    deg = degp[0, :, 0:1] + degp[1, :, 0:1]
    rid = i * br + lax.broadcasted_iota(jnp.int32, (br, 1), 0)
    deg = deg + jnp.where(rid < N, 1.0, 0.0).astype(F32)
    return jnp.where(deg > 0, lax.rsqrt(deg), 0.0)


def _mm_scale_body(x_ref, w_ref, degp_ref, o_ref):
    i = pl.program_id(0)
    dis = _dis_block(degp_ref[...], i, _BR)
    g = jnp.dot(x_ref[...], w_ref[...], preferred_element_type=F32) * dis
    o_ref[0] = g[:, :64]
    o_ref[1] = g[:, 64:]


def _tc_mm_scale(x_pad, W1, degp):
    return pl.pallas_call(
        _mm_scale_body,
        grid=(NPAD // _BR,),
        in_specs=[
            pl.BlockSpec((_BR, 128), lambda i: (i, 0)),
            pl.BlockSpec((128, 128), lambda i: (0, 0)),
            pl.BlockSpec((2, _BR, 16), lambda i: (0, i, 0)),
        ],
        out_specs=pl.BlockSpec((2, _BR, 64), lambda i: (0, i, 0)),
        out_shape=jax.ShapeDtypeStruct((2, NPAD, 64), F32),
    )(x_pad, W1, degp)


def _layer_body(p_ref, degp_ref, b_ref, w_ref, o_ref):
    i = pl.program_id(0)
    dis = _dis_block(degp_ref[...], i, _BR)
    ha = jnp.maximum(p_ref[0] * dis + b_ref[0, :, :64], 0.0)
    hb = jnp.maximum(p_ref[1] * dis + b_ref[0, :, 64:], 0.0)
    g2 = (jnp.dot(ha, w_ref[0], preferred_element_type=F32)
          + jnp.dot(hb, w_ref[1], preferred_element_type=F32)) * dis
    o_ref[0] = g2[:, :32]
    o_ref[1] = g2[:, 32:]


def _tc_layer(p1, degp, b1, W2):
    return pl.pallas_call(
        _layer_body,
        grid=(NPAD // _BR,),
        in_specs=[
            pl.BlockSpec((2, _BR, 64), lambda i: (0, i, 0)),
            pl.BlockSpec((2, _BR, 16), lambda i: (0, i, 0)),
            pl.BlockSpec((1, 1, 128), lambda i: (0, 0, 0)),
            pl.BlockSpec((2, 64, 64), lambda i: (0, 0, 0)),
        ],
        out_specs=pl.BlockSpec((2, _BR, 32), lambda i: (0, i, 0)),
        out_shape=jax.ShapeDtypeStruct((2, NPAD, 32), F32),
    )(p1, degp, b1, W2)


def _final_h_body(p_ref, degp_ref, b_ref, o_ref):
    i = pl.program_id(0)
    dis = _dis_block(degp_ref[...], i, _BR)
    o_ref[:, :32] = p_ref[0] * dis + b_ref[0, :, :32]
    o_ref[:, 32:] = p_ref[1] * dis + b_ref[0, :, 32:]


def _tc_final_h(p2, degp, b2):
    return pl.pallas_call(
        _final_h_body,
        grid=(NPAD // _BR,),
        in_specs=[
            pl.BlockSpec((2, _BR, 32), lambda i: (0, i, 0)),
            pl.BlockSpec((2, _BR, 16), lambda i: (0, i, 0)),
            pl.BlockSpec((1, 1, 64), lambda i: (0, 0, 0)),
        ],
        out_specs=pl.BlockSpec((_BR, 64), lambda i: (i, 0)),
        out_shape=jax.ShapeDtypeStruct((NPAD, 64), F32),
    )(p2, degp, b2)


_BM = 512
_BN = 4096


def _nt_body(a_ref, b_ref, o_ref):
    o_ref[...] = lax.dot_general(
        a_ref[...], b_ref[...], (((1,), (1,)), ((), ())),
        preferred_element_type=F32)


def _tc_decoder(rna, drug):
    gm = pl.cdiv(R, _BM)
    gn = pl.cdiv(N - R, _BN)
    return pl.pallas_call(
        _nt_body,
        grid=(gm, gn),
        in_specs=[
            pl.BlockSpec((_BM, 64), lambda i, j: (i, 0)),
            pl.BlockSpec((_BN, 64), lambda i, j: (j, 0)),
        ],
        out_specs=pl.BlockSpec((_BM, _BN), lambda i, j: (i, j)),
        out_shape=jax.ShapeDtypeStruct((R, N - R), F32),
    )(rna, drug)


def kernel(x, edge_index, W1, b1, W2, b2):
    ei = edge_index.astype(jnp.int32)
    padfill = jnp.full((EPAD - E,), N, jnp.int32)
    src3 = jnp.concatenate([ei[0], padfill]).reshape(NS, CH, K)
    dst3 = jnp.concatenate([ei[1], padfill]).reshape(NS, CH, K)
    x_pad = jnp.pad(x, ((0, NPAD - N), (0, 0)))

    degp = _sc_degree(dst3)
    g1 = _tc_mm_scale(x_pad, W1, degp)
    p1 = _sc_agg1(src3, dst3, g1)
    g2 = _tc_layer(p1, degp, b1.reshape(1, 1, 128), W2.reshape(2, 64, 64))
    p2 = _sc_agg2(src3, dst3, g2)
    h2 = _tc_final_h(p2, degp, b2.reshape(1, 1, 64))

    rna = h2[:R]
    drug = h2[R:N]
    return _tc_decoder(rna, drug)
